# Initial kernel scaffold; baseline (speedup 1.0000x reference)
import jax
import jax.numpy as jnp
from jax.experimental import pallas as pl


def kernel(temporal_dist, time_mask, frames_id):
    raise NotImplementedError("write your pallas kernel here")


# speedup vs baseline: 1.4564x; 1.4564x over previous
import functools

import jax
import jax.numpy as jnp
from jax import lax
from jax.experimental import pallas as pl
from jax.experimental.pallas import tpu as pltpu
from jax.experimental.pallas import tpu_sc as plsc

B = 16
T = 1024
L = 16
NCHUNK = T // L
NEG_INF = float("-inf")


def _scan_body(i, carry, vs, ve):
    best_val, best_e, best_s, run_val, run_idx = carry
    base = L + i * L
    s_vec = vs[pl.ds(base, L)]
    sh_vec = vs[pl.ds(base - 1, L)]
    e_vec = ve[pl.ds(i * L, L)]
    lane = lax.broadcasted_iota(jnp.int32, (L,), 0)

    pfx = jnp.maximum(plsc.cummax(sh_vec), run_val)
    cand = pfx + e_vec
    cmax = jnp.max(cand)
    upd = cmax > best_val

    lane_e = jnp.max(plsc.all_reduce_ffs(cand == cmax))
    p_val = jnp.max(jnp.where(lane == lane_e, pfx, NEG_INF))
    in_run = run_val >= p_val
    m_s = jnp.logical_and(s_vec == p_val, lane < lane_e)
    lane_s = jnp.max(plsc.all_reduce_ffs(m_s))
    s_cand = jnp.where(in_run, run_idx, i * L + lane_s)

    best_e = jnp.where(upd, i * L + lane_e, best_e)
    best_s = jnp.where(upd, s_cand, best_s)
    best_val = jnp.where(upd, cmax, best_val)

    smax = jnp.max(s_vec)
    supd = smax > run_val
    lane_sm = jnp.max(plsc.all_reduce_ffs(s_vec == smax))
    run_idx = jnp.where(supd, i * L + lane_sm, run_idx)
    run_val = jnp.where(supd, smax, run_val)
    return best_val, best_e, best_s, run_val, run_idx


def _sc_body(starts_hbm, ends_hbm, frames_hbm, out_hbm, vs, ve, vf, vout):
    cid = lax.axis_index("c")
    sid = lax.axis_index("s")

    @pl.when(cid == 0)
    def _():
        b = sid
        vs[pl.ds(0, L)] = jnp.full((L,), NEG_INF, jnp.float32)
        pltpu.sync_copy(starts_hbm.at[pl.ds(b * T, T)], vs.at[pl.ds(L, T)])
        pltpu.sync_copy(ends_hbm.at[pl.ds(b * T, T)], ve)
        pltpu.sync_copy(frames_hbm.at[pl.ds(b * T, T)], vf)

        init = (jnp.float32(NEG_INF), jnp.int32(0), jnp.int32(0),
                jnp.float32(NEG_INF), jnp.int32(0))
        body = functools.partial(_scan_body, vs=vs, ve=ve)
        _, best_e, best_s, _, _ = lax.fori_loop(0, NCHUNK, body, init)

        lane = lax.broadcasted_iota(jnp.int32, (L,), 0)
        idx = jnp.where(lane == 0, best_s, best_e)
        frames = plsc.load_gather(vf, [idx])
        res = frames.astype(jnp.float32) + jnp.where(
            lane == 1, jnp.float32(1.0), jnp.float32(0.0))
        vout[...] = res
        pltpu.sync_copy(vout, out_hbm.at[pl.ds(b * L, L)])


@jax.jit
def _post_process(starts, ends, frames):
    mesh = plsc.VectorSubcoreMesh(core_axis_name="c", subcore_axis_name="s")
    run = pl.kernel(
        _sc_body,
        out_type=jax.ShapeDtypeStruct((B * L,), jnp.float32),
        mesh=mesh,
        compiler_params=pltpu.CompilerParams(needs_layout_passes=False),
        scratch_types=[
            pltpu.VMEM((T + L,), jnp.float32),
            pltpu.VMEM((T,), jnp.float32),
            pltpu.VMEM((T,), jnp.int32),
            pltpu.VMEM((L,), jnp.float32),
        ],
    )
    out = run(starts, ends, frames)
    return out.reshape(B, L)[:, :2]


def kernel(temporal_dist, time_mask, frames_id):
    del time_mask
    starts = temporal_dist[:, :, 0].reshape(-1)
    ends = temporal_dist[:, :, 1].reshape(-1)
    frames = frames_id.reshape(-1).astype(jnp.int32)
    return _post_process(starts, ends, frames)
